# baseline (speedup 1.0000x reference)
import functools

import jax
import jax.numpy as jnp
from jax import lax
from jax.experimental import pallas as pl
from jax.experimental.pallas import tpu as pltpu
from jax.experimental.pallas import tpu_sc as plsc

B, N, C = 4, 4096, 128
S, K = 1024, 64
R2 = 0.2 * 0.2

NC, NS = 2, 16
NW = NC * NS
S_PER_W = (B * S) // NW
C_PER_W = (B * C) // NW
CHUNK = 4096

@functools.lru_cache(maxsize=None)
def _mesh():
    return plsc.VectorSubcoreMesh(core_axis_name="c", subcore_axis_name="s",
                                  num_cores=NC, num_subcores=NS)


FR, FC = 2 * B, N // 2
FCH = 512
NCH = FC // FCH


def _fps_body(x_ref, y_ref, z_ref, out_ref, dist_ref):
    rowhalf = (lax.broadcasted_iota(jnp.int32, (FR, FCH), 0) >> 2) * FC
    lanei = lax.broadcasted_iota(jnp.int32, (FR, FCH), 1)
    dist_ref[...] = jnp.full((FR, FC), 1e10, jnp.float32)
    init = (x_ref[0:B, 0:1], y_ref[0:B, 0:1], z_ref[0:B, 0:1])

    def step(s, carry):
        cx4, cy4, cz4 = carry
        row = jnp.concatenate([cx4, cy4, cz4, jnp.zeros_like(cx4)], axis=1)
        out_ref[pl.ds(s, 1), :, :] = row[None]
        cx = jnp.concatenate([cx4, cx4], axis=0)
        cy = jnp.concatenate([cy4, cy4], axis=0)
        cz = jnp.concatenate([cz4, cz4], axis=0)

        rm = jnp.full((FR, FCH), -1.0, jnp.float32)
        am = jnp.zeros((FR, FCH), jnp.int32)
        xm = jnp.zeros((FR, FCH), jnp.float32)
        ym = jnp.zeros((FR, FCH), jnp.float32)
        zm = jnp.zeros((FR, FCH), jnp.float32)
        for c in range(NCH):
            sl = pl.ds(c * FCH, FCH)
            xs = x_ref[:, sl]
            ys = y_ref[:, sl]
            zs = z_ref[:, sl]
            dx = xs - cx
            dy = ys - cy
            dz = zs - cz
            d = dx * dx + dy * dy + dz * dz
            dn = jnp.minimum(dist_ref[:, sl], d)
            dist_ref[:, sl] = dn
            upd = dn > rm
            linc = rowhalf + lanei + c * FCH
            am = jnp.where(upd, linc, am)
            xm = jnp.where(upd, xs, xm)
            ym = jnp.where(upd, ys, ym)
            zm = jnp.where(upd, zs, zm)
            rm = jnp.maximum(rm, dn)

        m8 = jnp.max(rm, axis=1, keepdims=True)
        m4 = jnp.maximum(m8[0:B], m8[B:FR])
        m = jnp.concatenate([m4, m4], axis=0)
        cand = jnp.where(rm == m, am, jnp.int32(N))
        i8 = jnp.min(cand, axis=1, keepdims=True)
        i4 = jnp.minimum(i8[0:B], i8[B:FR])
        idxb = jnp.concatenate([i4, i4], axis=0)
        sel = am == idxb
        nx8 = jnp.sum(jnp.where(sel, xm, 0.0), axis=1, keepdims=True)
        ny8 = jnp.sum(jnp.where(sel, ym, 0.0), axis=1, keepdims=True)
        nz8 = jnp.sum(jnp.where(sel, zm, 0.0), axis=1, keepdims=True)
        return (nx8[0:B] + nx8[B:FR], ny8[0:B] + ny8[B:FR],
                nz8[0:B] + nz8[B:FR])

    lax.fori_loop(0, S, step, init)


_fps_call = pl.pallas_call(
    _fps_body,
    out_shape=jax.ShapeDtypeStruct((S, B, 4), jnp.float32),
    scratch_shapes=[pltpu.VMEM((FR, FC), jnp.float32)],
)


CSTR = 128
NG = S_PER_W // 16


def _bq_body(x_hbm, y_hbm, z_hbm, cx_hbm, cy_hbm, cz_hbm, idx_hbm, gxyz_hbm,
             xv, yv, zv, cxv, cyv, czv, cbuf, idx_stage, xyz_stage):
    wid = lax.axis_index("s") * NC + lax.axis_index("c")
    wpb = NW // B
    b = wid // wpb
    s0 = (wid % wpb) * S_PER_W
    pltpu.sync_copy(x_hbm.at[b], xv)
    pltpu.sync_copy(y_hbm.at[b], yv)
    pltpu.sync_copy(z_hbm.at[b], zv)
    pltpu.sync_copy(cx_hbm.at[b, pl.ds(s0, S_PER_W)], cxv)
    pltpu.sync_copy(cy_hbm.at[b, pl.ds(s0, S_PER_W)], cyv)
    pltpu.sync_copy(cz_hbm.at[b, pl.ds(s0, S_PER_W)], czv)
    lane = jax.lax.broadcasted_iota(jnp.int32, (16,), 0)
    lane_s = lane * CSTR

    def per_group(g, _):
        cxg = cxv[pl.ds(g * 16, 16)]
        cyg = cyv[pl.ds(g * 16, 16)]
        czg = czv[pl.ds(g * 16, 16)]

        def scan(i, ptrs):
            xvv = xv[pl.ds(i * 16, 16)]
            yvv = yv[pl.ds(i * 16, 16)]
            zvv = zv[pl.ds(i * 16, 16)]
            for j in range(16):
                dx = cxg - xvv[j]
                dy = cyg - yvv[j]
                dz = czg - zvv[j]
                d = dx * dx + dy * dy + dz * dz
                msk = d <= R2
                addr = lane_s + jnp.minimum(ptrs, CSTR - 2)
                plsc.store_scatter(cbuf, [addr],
                                   jnp.full((16,), i * 16 + j, jnp.int32),
                                   mask=msk)
                ptrs = ptrs + msk.astype(jnp.int32)
            return ptrs

        ptrs = lax.fori_loop(0, N // 16, scan, jnp.zeros((16,), jnp.int32))

        for t in range(16):
            count = jnp.minimum(ptrs[t], K)
            first = cbuf[pl.ds(t * CSTR, 16)][0]
            cx = cxg[t]
            cy = cyg[t]
            cz = czg[t]
            base = (g * 16 + t) * K
            for kk in range(4):
                iv = cbuf[pl.ds(t * CSTR + kk * 16, 16)]
                sel = (lane + kk * 16) < count
                iv = jnp.where(sel, iv, first)
                gx = plsc.load_gather(xv, [iv]) - cx
                gy = plsc.load_gather(yv, [iv]) - cy
                gz = plsc.load_gather(zv, [iv]) - cz
                idx_stage[pl.ds(base + kk * 16, 16)] = iv
                xyz_stage[0, pl.ds(base + kk * 16, 16)] = gx
                xyz_stage[1, pl.ds(base + kk * 16, 16)] = gy
                xyz_stage[2, pl.ds(base + kk * 16, 16)] = gz
        return 0

    lax.fori_loop(0, NG, per_group, 0)
    pltpu.sync_copy(idx_stage, idx_hbm.at[b, pl.ds(s0 * K, S_PER_W * K)])
    pltpu.sync_copy(xyz_stage, gxyz_hbm.at[b, :, pl.ds(s0 * K, S_PER_W * K)])


@functools.lru_cache(maxsize=None)
def _bq_call():
    return pl.kernel(
        _bq_body,
        out_type=(jax.ShapeDtypeStruct((B, S * K), jnp.int32),
                  jax.ShapeDtypeStruct((B, 3, S * K), jnp.float32)),
        mesh=_mesh(),
        compiler_params=pltpu.CompilerParams(needs_layout_passes=False),
        scratch_types=[
            pltpu.VMEM((N,), jnp.float32),
            pltpu.VMEM((N,), jnp.float32),
            pltpu.VMEM((N,), jnp.float32),
            pltpu.VMEM((S_PER_W,), jnp.float32),
            pltpu.VMEM((S_PER_W,), jnp.float32),
            pltpu.VMEM((S_PER_W,), jnp.float32),
            pltpu.VMEM((16 * CSTR,), jnp.int32),
            pltpu.VMEM((S_PER_W * K,), jnp.int32),
            pltpu.VMEM((3, S_PER_W * K), jnp.float32),
        ],
    )


HALF = CHUNK // 2


XB = (S * K) // (NW // B)
ROW = S * K


def _gather_body(feat_hbm, idx_hbm, gxyz_hbm, out_hbm,
                 tab, idxv, ostage, xbuf, osem, isem):
    wid = lax.axis_index("s") * NC + lax.axis_index("c")
    wpb = NW // B
    b = wid // wpb
    cw = wid % wpb
    c0 = cw * C_PER_W
    obase = b * (3 + C) * ROW
    pltpu.sync_copy(feat_hbm.at[b, pl.ds(c0 * N, C_PER_W * N)], tab)

    def _copy_xyz(r, _):
        pltpu.sync_copy(gxyz_hbm.at[b, r, pl.ds(cw * XB, XB)], xbuf)
        pltpu.sync_copy(xbuf, out_hbm.at[pl.ds(obase + r * ROW + cw * XB, XB)])
        return 0

    lax.fori_loop(0, 3, _copy_xyz, 0)

    n_chunks = ROW // CHUNK
    pltpu.sync_copy(idx_hbm.at[b, pl.ds(0, CHUNK)], idxv.at[0])

    def per_chunk(ch, _):
        cbuf = ch % 2

        @pl.when(ch + 1 < n_chunks)
        def _prefetch():
            pltpu.async_copy(idx_hbm.at[b, pl.ds((ch + 1) * CHUNK, CHUNK)],
                             idxv.at[1 - cbuf], isem)

        def per_row(r, _):
            t = ch * C_PER_W + r
            obuf = t % 2

            @pl.when(t >= 2)
            def _drain():
                pltpu.make_async_copy(idx_hbm.at[b, pl.ds(0, CHUNK)],
                                      ostage.at[obuf], osem).wait()

            roff = r * N

            @plsc.parallel_loop(0, CHUNK, step=16, unroll=8)
            def gat(j):
                iv = idxv[cbuf, pl.ds(j, 16)] + roff
                ostage[obuf, pl.ds(j, 16)] = plsc.load_gather(tab, [iv])
            pltpu.async_copy(
                ostage.at[obuf],
                out_hbm.at[pl.ds(obase + (3 + c0 + r) * ROW + ch * CHUNK,
                                 CHUNK)],
                osem)
            return 0

        lax.fori_loop(0, C_PER_W, per_row, 0)

        @pl.when(ch + 1 < n_chunks)
        def _wait_prefetch():
            pltpu.make_async_copy(idx_hbm.at[b, pl.ds(0, CHUNK)],
                                  idxv.at[1 - cbuf], isem).wait()

        return 0

    lax.fori_loop(0, n_chunks, per_chunk, 0)
    pltpu.make_async_copy(idx_hbm.at[b, pl.ds(0, CHUNK)],
                          ostage.at[0], osem).wait()
    pltpu.make_async_copy(idx_hbm.at[b, pl.ds(0, CHUNK)],
                          ostage.at[1], osem).wait()


@functools.lru_cache(maxsize=None)
def _gather_call():
    return pl.kernel(
        _gather_body,
        out_type=jax.ShapeDtypeStruct((B * (3 + C) * ROW,), jnp.float32),
        mesh=_mesh(),
        compiler_params=pltpu.CompilerParams(needs_layout_passes=False),
        scratch_types=[
            pltpu.VMEM((C_PER_W * N,), jnp.float32),
            pltpu.VMEM((2, CHUNK), jnp.int32),
            pltpu.VMEM((2, CHUNK), jnp.float32),
            pltpu.VMEM((XB,), jnp.float32),
            pltpu.SemaphoreType.DMA,
            pltpu.SemaphoreType.DMA,
        ],
    )


def kernel(point_coord, features):
    x = point_coord[..., 0]
    y = point_coord[..., 1]
    z = point_coord[..., 2]
    def _fold(a):
        return a.reshape(B, 2, FC).transpose(1, 0, 2).reshape(FR, FC)

    fps_out = _fps_call(_fold(x), _fold(y), _fold(z))
    cent = jnp.transpose(fps_out, (1, 0, 2))
    idx, gxyz = _bq_call()(x, y, z, cent[..., 0], cent[..., 1], cent[..., 2])
    out = _gather_call()(features.reshape(B, C * N), idx, gxyz)
    return out.reshape(B, 3 + C, S, K)


# speedup vs baseline: 1.2279x; 1.2279x over previous
import functools

import jax
import jax.numpy as jnp
from jax import lax
from jax.experimental import pallas as pl
from jax.experimental.pallas import tpu as pltpu
from jax.experimental.pallas import tpu_sc as plsc

B, N, C = 4, 4096, 128
S, K = 1024, 64
R2 = 0.2 * 0.2

NC, NS = 2, 16
NW = NC * NS
S_PER_W = (B * S) // NW
C_PER_W = (B * C) // NW
CHUNK = 4096

@functools.lru_cache(maxsize=None)
def _mesh():
    return plsc.VectorSubcoreMesh(core_axis_name="c", subcore_axis_name="s",
                                  num_cores=NC, num_subcores=NS)


def _fps_body(x_ref, y_ref, z_ref, out_ref):
    x = x_ref[...]
    y = y_ref[...]
    z = z_ref[...]
    lin = lax.broadcasted_iota(jnp.int32, (B, N), 1)

    def step(s, carry):
        dist, cx, cy, cz = carry
        row = jnp.concatenate([cx, cy, cz, jnp.zeros_like(cx)], axis=1)
        out_ref[pl.ds(s, 1), :, :] = row[None]
        dx = x - cx
        dy = y - cy
        dz = z - cz
        d = dx * dx + dy * dy + dz * dz
        dist = jnp.minimum(dist, d)
        m = jnp.max(dist, axis=1, keepdims=True)
        idx = jnp.min(jnp.where(dist == m, lin, N), axis=1, keepdims=True)
        sel = lin == idx
        ncx = jnp.sum(jnp.where(sel, x, 0.0), axis=1, keepdims=True)
        ncy = jnp.sum(jnp.where(sel, y, 0.0), axis=1, keepdims=True)
        ncz = jnp.sum(jnp.where(sel, z, 0.0), axis=1, keepdims=True)
        return dist, ncx, ncy, ncz

    init = (jnp.full((B, N), 1e10, jnp.float32), x[:, :1], y[:, :1], z[:, :1])
    lax.fori_loop(0, S, step, init)


_fps_call = pl.pallas_call(
    _fps_body,
    out_shape=jax.ShapeDtypeStruct((S, B, 4), jnp.float32),
)


CSTR = 128
NG = S_PER_W // 16


def _bq_body(x_hbm, y_hbm, z_hbm, cx_hbm, cy_hbm, cz_hbm, idx_hbm, gxyz_hbm,
             xv, yv, zv, cxv, cyv, czv, cbuf, idx_stage, xyz_stage):
    wid = lax.axis_index("s") * NC + lax.axis_index("c")
    wpb = NW // B
    b = wid // wpb
    s0 = (wid % wpb) * S_PER_W
    pltpu.sync_copy(x_hbm.at[b], xv)
    pltpu.sync_copy(y_hbm.at[b], yv)
    pltpu.sync_copy(z_hbm.at[b], zv)
    pltpu.sync_copy(cx_hbm.at[b, pl.ds(s0, S_PER_W)], cxv)
    pltpu.sync_copy(cy_hbm.at[b, pl.ds(s0, S_PER_W)], cyv)
    pltpu.sync_copy(cz_hbm.at[b, pl.ds(s0, S_PER_W)], czv)
    lane = jax.lax.broadcasted_iota(jnp.int32, (16,), 0)
    lane_s = lane * CSTR

    def per_group(g, _):
        cxg = cxv[pl.ds(g * 16, 16)]
        cyg = cyv[pl.ds(g * 16, 16)]
        czg = czv[pl.ds(g * 16, 16)]

        def scan(i, ptrs):
            xvv = xv[pl.ds(i * 16, 16)]
            yvv = yv[pl.ds(i * 16, 16)]
            zvv = zv[pl.ds(i * 16, 16)]
            for j in range(16):
                dx = cxg - xvv[j]
                dy = cyg - yvv[j]
                dz = czg - zvv[j]
                d = dx * dx + dy * dy + dz * dz
                msk = d <= R2
                addr = lane_s + jnp.minimum(ptrs, CSTR - 2)
                plsc.store_scatter(cbuf, [addr],
                                   jnp.full((16,), i * 16 + j, jnp.int32),
                                   mask=msk)
                ptrs = ptrs + msk.astype(jnp.int32)
            return ptrs

        ptrs = lax.fori_loop(0, N // 16, scan, jnp.zeros((16,), jnp.int32))

        for t in range(16):
            count = jnp.minimum(ptrs[t], K)
            first = cbuf[pl.ds(t * CSTR, 16)][0]
            cx = cxg[t]
            cy = cyg[t]
            cz = czg[t]
            sl_pos = g * 16 + t
            for kk in range(4):
                iv = cbuf[pl.ds(t * CSTR + kk * 16, 16)]
                sel = (lane + kk * 16) < count
                iv = jnp.where(sel, iv, first)
                gx = plsc.load_gather(xv, [iv]) - cx
                gy = plsc.load_gather(yv, [iv]) - cy
                gz = plsc.load_gather(zv, [iv]) - cz
                kv = lane + kk * 16
                sv = jnp.full((16,), sl_pos, jnp.int32)
                plsc.store_scatter(idx_stage, [kv, sv], iv)
                zero = jnp.zeros((16,), jnp.int32)
                plsc.store_scatter(xyz_stage, [zero, kv, sv], gx)
                plsc.store_scatter(xyz_stage, [zero + 1, kv, sv], gy)
                plsc.store_scatter(xyz_stage, [zero + 2, kv, sv], gz)
        return 0

    lax.fori_loop(0, NG, per_group, 0)
    pltpu.sync_copy(idx_stage, idx_hbm.at[b, :, pl.ds(s0, S_PER_W)])
    pltpu.sync_copy(xyz_stage, gxyz_hbm.at[b, :, :, pl.ds(s0, S_PER_W)])


@functools.lru_cache(maxsize=None)
def _bq_call():
    return pl.kernel(
        _bq_body,
        out_type=(jax.ShapeDtypeStruct((B, K, S), jnp.int32),
                  jax.ShapeDtypeStruct((B, 3, K, S), jnp.float32)),
        mesh=_mesh(),
        compiler_params=pltpu.CompilerParams(needs_layout_passes=False),
        scratch_types=[
            pltpu.VMEM((N,), jnp.float32),
            pltpu.VMEM((N,), jnp.float32),
            pltpu.VMEM((N,), jnp.float32),
            pltpu.VMEM((S_PER_W,), jnp.float32),
            pltpu.VMEM((S_PER_W,), jnp.float32),
            pltpu.VMEM((S_PER_W,), jnp.float32),
            pltpu.VMEM((16 * CSTR,), jnp.int32),
            pltpu.VMEM((K, S_PER_W), jnp.int32),
            pltpu.VMEM((3, K, S_PER_W), jnp.float32),
        ],
    )


HALF = CHUNK // 2


XB = (S * K) // (NW // B)
ROW = S * K


def _gather_body(feat_hbm, idx_hbm, gxyz_hbm, out_hbm,
                 tab, idxv, ostage, xbuf, osem, isem):
    wid = lax.axis_index("s") * NC + lax.axis_index("c")
    wpb = NW // B
    b = wid // wpb
    cw = wid % wpb
    c0 = cw * C_PER_W
    obase = b * (3 + C) * ROW
    pltpu.sync_copy(feat_hbm.at[b, pl.ds(c0 * N, C_PER_W * N)], tab)

    def _copy_xyz(r, _):
        pltpu.sync_copy(gxyz_hbm.at[b, r, pl.ds(cw * XB, XB)], xbuf)
        pltpu.sync_copy(xbuf, out_hbm.at[pl.ds(obase + r * ROW + cw * XB, XB)])
        return 0

    lax.fori_loop(0, 3, _copy_xyz, 0)

    n_chunks = ROW // CHUNK
    pltpu.sync_copy(idx_hbm.at[b, pl.ds(0, CHUNK)], idxv.at[0])

    def per_chunk(ch, _):
        cbuf = ch % 2

        @pl.when(ch + 1 < n_chunks)
        def _prefetch():
            pltpu.async_copy(idx_hbm.at[b, pl.ds((ch + 1) * CHUNK, CHUNK)],
                             idxv.at[1 - cbuf], isem)

        def per_row(r, _):
            t = ch * C_PER_W + r
            obuf = t % 2

            @pl.when(t >= 2)
            def _drain():
                pltpu.make_async_copy(idx_hbm.at[b, pl.ds(0, CHUNK)],
                                      ostage.at[obuf], osem).wait()

            roff = r * N

            @plsc.parallel_loop(0, CHUNK, step=16, unroll=8)
            def gat(j):
                iv = idxv[cbuf, pl.ds(j, 16)] + roff
                ostage[obuf, pl.ds(j, 16)] = plsc.load_gather(tab, [iv])
            pltpu.async_copy(
                ostage.at[obuf],
                out_hbm.at[pl.ds(obase + (3 + c0 + r) * ROW + ch * CHUNK,
                                 CHUNK)],
                osem)
            return 0

        lax.fori_loop(0, C_PER_W, per_row, 0)

        @pl.when(ch + 1 < n_chunks)
        def _wait_prefetch():
            pltpu.make_async_copy(idx_hbm.at[b, pl.ds(0, CHUNK)],
                                  idxv.at[1 - cbuf], isem).wait()

        return 0

    lax.fori_loop(0, n_chunks, per_chunk, 0)
    pltpu.make_async_copy(idx_hbm.at[b, pl.ds(0, CHUNK)],
                          ostage.at[0], osem).wait()
    pltpu.make_async_copy(idx_hbm.at[b, pl.ds(0, CHUNK)],
                          ostage.at[1], osem).wait()


@functools.lru_cache(maxsize=None)
def _gather_call():
    return pl.kernel(
        _gather_body,
        out_type=jax.ShapeDtypeStruct((B * (3 + C) * ROW,), jnp.float32),
        mesh=_mesh(),
        compiler_params=pltpu.CompilerParams(needs_layout_passes=False),
        scratch_types=[
            pltpu.VMEM((C_PER_W * N,), jnp.float32),
            pltpu.VMEM((2, CHUNK), jnp.int32),
            pltpu.VMEM((2, CHUNK), jnp.float32),
            pltpu.VMEM((XB,), jnp.float32),
            pltpu.SemaphoreType.DMA,
            pltpu.SemaphoreType.DMA,
        ],
    )


def kernel(point_coord, features):
    x = point_coord[..., 0]
    y = point_coord[..., 1]
    z = point_coord[..., 2]
    fps_out = _fps_call(x, y, z)
    cent = jnp.transpose(fps_out, (1, 0, 2))
    idx, gxyz = _bq_call()(x, y, z, cent[..., 0], cent[..., 1], cent[..., 2])
    out = _gather_call()(features.reshape(B, C * N),
                         idx.reshape(B, K * S),
                         gxyz.reshape(B, 3, K * S))
    return jnp.transpose(out.reshape(B, 3 + C, K, S), (0, 1, 3, 2))
